# baseline (speedup 1.0000x reference)
import functools

import jax
import jax.numpy as jnp
from jax import lax
from jax.experimental import pallas as pl
from jax.experimental.pallas import tpu as pltpu
from jax.experimental.pallas import tpu_sc as plsc

_B, _S = 4, 4096
_NW = 32
_CW = _S // _NW
_LANES = 16
_KI = _CW // _LANES
_NL = _B * _S * 2
_TCR, _TCC = _NL // 128, 128


def _threefry_bits(seed, idx_u32):
    ks0 = jnp.uint32(0)
    ks1 = jnp.uint32(seed)
    ks2 = ks0 ^ ks1 ^ jnp.uint32(0x1BD11BDA)
    ks = (ks0, ks1, ks2)
    rotations = ((13, 15, 26, 6), (17, 29, 16, 24))
    x0 = jnp.zeros_like(idx_u32) + ks0
    x1 = idx_u32 + ks1
    for i in range(5):
        for r in rotations[i % 2]:
            x0 = x0 + x1
            x1 = (x1 << jnp.uint32(r)) | (x1 >> jnp.uint32(32 - r))
            x1 = x0 ^ x1
        x0 = x0 + ks[(i + 1) % 3]
        x1 = x1 + ks[(i + 2) % 3] + jnp.uint32(i + 1)
    return x0 ^ x1


def _to_uniform(bits):
    fb = (bits >> jnp.uint32(9)) | jnp.uint32(0x3F800000)
    return lax.bitcast_convert_type(fb, jnp.float32) - jnp.float32(1.0)


def _uniform_at(seed, idx_i32):
    return _to_uniform(
        _threefry_bits(seed, lax.bitcast_convert_type(idx_i32, jnp.uint32)))


def _step(j_base, lane, xv):
    j = j_base + lane
    l0 = _uniform_at(1, j * 2)
    l1 = _uniform_at(1, j * 2 + 1)
    um = _uniform_at(2, j)
    m = jnp.maximum(l0, l1)
    e0 = jnp.exp(l0 - m)
    e1 = jnp.exp(l1 - m)
    s = e0 + e1
    mk = um < jnp.float32(0.5)
    mask01 = jnp.where(mk, jnp.full((_LANES,), 1, jnp.int32),
                       jnp.full((_LANES,), 0, jnp.int32))
    chosen = jnp.where(mk, e1 / s, e0 / s)
    masked = jnp.where(mk, xv, jnp.zeros((_LANES,), jnp.float32))
    return chosen, mask01, masked


@functools.cache
def _build_sc_kernel():
    mesh = plsc.VectorSubcoreMesh(core_axis_name="c", subcore_axis_name="s")
    f32 = jnp.float32

    @functools.partial(
        pl.kernel,
        mesh=mesh,
        out_type=[
            jax.ShapeDtypeStruct((_B, _S), f32),
            jax.ShapeDtypeStruct((_B, _S), jnp.int32),
            jax.ShapeDtypeStruct((_B, _S), f32),
        ],
        scratch_types=[
            pltpu.VMEM((_B, _CW), f32),
            pltpu.VMEM((_B, _CW), f32),
            pltpu.VMEM((_B, _CW), jnp.int32),
            pltpu.VMEM((_B, _CW), f32),
            pltpu.SemaphoreType.DMA,
        ],
    )
    def _sc_kernel(x_hbm, ch_hbm, m_hbm, mi_hbm,
                   x_v, ch_v, m_v, mi_v, sem):
        wid = lax.axis_index("s") * 2 + lax.axis_index("c")
        cb = wid * _CW
        csl = pl.ds(cb, _CW)
        pltpu.sync_copy(x_hbm.at[:, csl], x_v)

        def body(k, carry):
            lane = lax.iota(jnp.int32, _LANES)
            o = k * _LANES
            sl = pl.ds(o, _LANES)
            for b in range(_B):
                xv = x_v[b, sl]
                ch, m01, masked = _step(b * _S + cb + o, lane, xv)
                ch_v[b, sl] = ch
                m_v[b, sl] = m01
                mi_v[b, sl] = masked
            return carry

        lax.fori_loop(0, _KI, body, 0)

        copies = [
            pltpu.async_copy(ch_v, ch_hbm.at[:, csl], sem),
            pltpu.async_copy(m_v, m_hbm.at[:, csl], sem),
            pltpu.async_copy(mi_v, mi_hbm.at[:, csl], sem),
        ]
        for c in copies:
            c.wait()

    return _sc_kernel


def _tc_body(l_ref, p_ref):
    r = lax.broadcasted_iota(jnp.int32, (_TCR, _TCC), 0)
    c = lax.broadcasted_iota(jnp.int32, (_TCR, _TCC), 1)
    f = r * _TCC + c
    u = _uniform_at(1, f)
    up = _uniform_at(1, f ^ 1)
    m = jnp.maximum(u, up)
    e = jnp.exp(u - m)
    ep = jnp.exp(up - m)
    l_ref[...] = u
    p_ref[...] = e / (e + ep)


@functools.cache
def _build_tc_kernel():
    return pl.pallas_call(
        _tc_body,
        out_shape=[
            jax.ShapeDtypeStruct((_TCR, _TCC), jnp.float32),
            jax.ShapeDtypeStruct((_TCR, _TCC), jnp.float32),
        ],
    )


def kernel(x):
    B, S = x.shape
    li, pi = _build_tc_kernel()()
    ch, m01, mi = _build_sc_kernel()(x)
    logits = li.reshape(B, S, 2)
    policy = pi.reshape(B, S, 2)
    chosen = ch.reshape(B, S, 1)
    mask = m01.astype(bool)
    masked = mi
    return logits, policy, chosen, mask, masked


# speedup vs baseline: 2.0093x; 2.0093x over previous
import functools

import jax
import jax.numpy as jnp
from jax import lax
from jax.experimental import pallas as pl
from jax.experimental.pallas import tpu as pltpu
from jax.experimental.pallas import tpu_sc as plsc

_B, _S = 4, 4096
_NW = 32
_CW = _S // _NW
_LANES = 16
_KI = _CW // _LANES
_NL = _B * _S * 2
_TCR, _TCC = _NL // 128, 128


def _threefry_bits(seed, idx_u32):
    ks0 = jnp.uint32(0)
    ks1 = jnp.uint32(seed)
    ks2 = ks0 ^ ks1 ^ jnp.uint32(0x1BD11BDA)
    ks = (ks0, ks1, ks2)
    rotations = ((13, 15, 26, 6), (17, 29, 16, 24))
    x0 = jnp.zeros_like(idx_u32) + ks0
    x1 = idx_u32 + ks1
    for i in range(5):
        for r in rotations[i % 2]:
            x0 = x0 + x1
            x1 = (x1 << jnp.uint32(r)) | (x1 >> jnp.uint32(32 - r))
            x1 = x0 ^ x1
        x0 = x0 + ks[(i + 1) % 3]
        x1 = x1 + ks[(i + 2) % 3] + jnp.uint32(i + 1)
    return x0 ^ x1


def _to_uniform(bits):
    fb = (bits >> jnp.uint32(9)) | jnp.uint32(0x3F800000)
    return lax.bitcast_convert_type(fb, jnp.float32) - jnp.float32(1.0)


def _uniform_at(seed, idx_i32):
    return _to_uniform(
        _threefry_bits(seed, lax.bitcast_convert_type(idx_i32, jnp.uint32)))


def _step(j_base, lane, xv):
    j = j_base + lane
    l0 = _uniform_at(1, j * 2)
    l1 = _uniform_at(1, j * 2 + 1)
    um = _uniform_at(2, j)
    m = jnp.maximum(l0, l1)
    e0 = jnp.exp(l0 - m)
    e1 = jnp.exp(l1 - m)
    s = e0 + e1
    mk = um < jnp.float32(0.5)
    mask01 = jnp.where(mk, jnp.full((_LANES,), 1, jnp.int32),
                       jnp.full((_LANES,), 0, jnp.int32))
    chosen = jnp.where(mk, e1 / s, e0 / s)
    masked = jnp.where(mk, xv, jnp.zeros((_LANES,), jnp.float32))
    return chosen, mask01, masked


@functools.cache
def _build_sc_kernel():
    mesh = plsc.VectorSubcoreMesh(core_axis_name="c", subcore_axis_name="s")
    f32 = jnp.float32

    @functools.partial(
        pl.kernel,
        mesh=mesh,
        out_type=[
            jax.ShapeDtypeStruct((_B, _S), f32),
            jax.ShapeDtypeStruct((_B, _S), jnp.int32),
            jax.ShapeDtypeStruct((_B, _S), f32),
        ],
        scratch_types=[
            pltpu.VMEM((_B, _CW), f32),
            pltpu.VMEM((_B, _CW), f32),
            pltpu.VMEM((_B, _CW), jnp.int32),
            pltpu.VMEM((_B, _CW), f32),
            pltpu.SemaphoreType.DMA,
        ],
    )
    def _sc_kernel(x_hbm, ch_hbm, m_hbm, mi_hbm,
                   x_v, ch_v, m_v, mi_v, sem):
        wid = lax.axis_index("s") * 2 + lax.axis_index("c")
        cb = wid * _CW
        csl = pl.ds(cb, _CW)
        pltpu.sync_copy(x_hbm.at[:, csl], x_v)

        def body(k, carry):
            lane = lax.iota(jnp.int32, _LANES)
            o = k * _LANES
            sl = pl.ds(o, _LANES)
            for b in range(_B):
                xv = x_v[b, sl]
                ch, m01, masked = _step(b * _S + cb + o, lane, xv)
                ch_v[b, sl] = ch
                m_v[b, sl] = m01
                mi_v[b, sl] = masked
            return carry

        lax.fori_loop(0, _KI, body, 0)

        copies = [
            pltpu.async_copy(ch_v, ch_hbm.at[:, csl], sem),
            pltpu.async_copy(m_v, m_hbm.at[:, csl], sem),
            pltpu.async_copy(mi_v, mi_hbm.at[:, csl], sem),
        ]
        for c in copies:
            c.wait()

    return _sc_kernel


def _tc_body(l0_ref, l1_ref, p0_ref, p1_ref):
    b = lax.broadcasted_iota(jnp.int32, (_B, _S), 0)
    s = lax.broadcasted_iota(jnp.int32, (_B, _S), 1)
    j = b * _S + s
    l0 = _uniform_at(1, j * 2)
    l1 = _uniform_at(1, j * 2 + 1)
    m = jnp.maximum(l0, l1)
    e0 = jnp.exp(l0 - m)
    e1 = jnp.exp(l1 - m)
    den = e0 + e1
    l0_ref[...] = l0
    l1_ref[...] = l1
    p0_ref[...] = e0 / den
    p1_ref[...] = e1 / den


@functools.cache
def _build_tc_kernel():
    plane = jax.ShapeDtypeStruct((_B, _S), jnp.float32)
    return pl.pallas_call(
        _tc_body,
        out_shape=[plane, plane, plane, plane],
    )


def kernel(x):
    B, S = x.shape
    l0, l1, p0, p1 = _build_tc_kernel()()
    ch, m01, mi = _build_sc_kernel()(x)
    logits = jnp.stack([l0, l1], axis=-1)
    policy = jnp.stack([p0, p1], axis=-1)
    chosen = ch.reshape(B, S, 1)
    mask = m01.astype(bool)
    masked = mi
    return logits, policy, chosen, mask, masked


# speedup vs baseline: 2.2032x; 1.0965x over previous
import functools

import jax
import jax.numpy as jnp
from jax import lax
from jax.experimental import pallas as pl
from jax.experimental.pallas import tpu as pltpu
from jax.experimental.pallas import tpu_sc as plsc

_B, _S = 4, 4096
_NW = 32
_CW = _S // _NW
_LANES = 16
_KI = _CW // _LANES
_NL = _B * _S * 2
_TCR, _TCC = _NL // 128, 128


def _threefry_bits(seed, idx_u32):
    ks0 = jnp.uint32(0)
    ks1 = jnp.uint32(seed)
    ks2 = ks0 ^ ks1 ^ jnp.uint32(0x1BD11BDA)
    ks = (ks0, ks1, ks2)
    rotations = ((13, 15, 26, 6), (17, 29, 16, 24))
    x0 = jnp.zeros_like(idx_u32) + ks0
    x1 = idx_u32 + ks1
    for i in range(5):
        for r in rotations[i % 2]:
            x0 = x0 + x1
            x1 = (x1 << jnp.uint32(r)) | (x1 >> jnp.uint32(32 - r))
            x1 = x0 ^ x1
        x0 = x0 + ks[(i + 1) % 3]
        x1 = x1 + ks[(i + 2) % 3] + jnp.uint32(i + 1)
    return x0 ^ x1


def _to_uniform(bits):
    fb = (bits >> jnp.uint32(9)) | jnp.uint32(0x3F800000)
    return lax.bitcast_convert_type(fb, jnp.float32) - jnp.float32(1.0)


def _uniform_at(seed, idx_i32):
    return _to_uniform(
        _threefry_bits(seed, lax.bitcast_convert_type(idx_i32, jnp.uint32)))


def _step(j_base, lane, xv, p0v, p1v):
    j = j_base + lane
    um = _uniform_at(2, j)
    mk = um < jnp.float32(0.5)
    chosen = jnp.where(mk, p1v, p0v)
    masked = jnp.where(mk, xv, jnp.zeros((_LANES,), jnp.float32))
    return chosen, masked


@functools.cache
def _build_sc_kernel():
    mesh = plsc.VectorSubcoreMesh(core_axis_name="c", subcore_axis_name="s")
    f32 = jnp.float32

    @functools.partial(
        pl.kernel,
        mesh=mesh,
        out_type=[
            jax.ShapeDtypeStruct((_B, _S), f32),
            jax.ShapeDtypeStruct((_B, _S), f32),
        ],
        scratch_types=[
            pltpu.VMEM((_B, _CW), f32),
            pltpu.VMEM((_B, _CW), f32),
            pltpu.VMEM((_B, _CW), f32),
            pltpu.VMEM((_B, _CW), f32),
            pltpu.VMEM((_B, _CW), f32),
            pltpu.SemaphoreType.DMA,
            pltpu.SemaphoreType.DMA,
        ],
    )
    def _sc_kernel(x_hbm, p0_hbm, p1_hbm, ch_hbm, mi_hbm,
                   x_v, p0_v, p1_v, ch_v, mi_v, sem, sem_in):
        wid = lax.axis_index("s") * 2 + lax.axis_index("c")
        cb = wid * _CW
        csl = pl.ds(cb, _CW)
        loads = [
            pltpu.async_copy(x_hbm.at[:, csl], x_v, sem_in),
            pltpu.async_copy(p0_hbm.at[:, csl], p0_v, sem_in),
            pltpu.async_copy(p1_hbm.at[:, csl], p1_v, sem_in),
        ]
        for c in loads:
            c.wait()

        def body(k, carry):
            lane = lax.iota(jnp.int32, _LANES)
            o = k * _LANES
            sl = pl.ds(o, _LANES)
            for b in range(_B):
                ch, masked = _step(b * _S + cb + o, lane,
                                   x_v[b, sl], p0_v[b, sl], p1_v[b, sl])
                ch_v[b, sl] = ch
                mi_v[b, sl] = masked
            return carry

        lax.fori_loop(0, _KI, body, 0)

        copies = [
            pltpu.async_copy(ch_v, ch_hbm.at[:, csl], sem),
            pltpu.async_copy(mi_v, mi_hbm.at[:, csl], sem),
        ]
        for c in copies:
            c.wait()

    return _sc_kernel


def _tc_body(l0_ref, l1_ref, p0_ref, p1_ref, mk_ref):
    b = lax.broadcasted_iota(jnp.int32, (_B, _S), 0)
    s = lax.broadcasted_iota(jnp.int32, (_B, _S), 1)
    j = b * _S + s
    l0 = _uniform_at(1, j * 2)
    l1 = _uniform_at(1, j * 2 + 1)
    m = jnp.maximum(l0, l1)
    e0 = jnp.exp(l0 - m)
    e1 = jnp.exp(l1 - m)
    den = e0 + e1
    l0_ref[...] = l0
    l1_ref[...] = l1
    p0_ref[...] = e0 / den
    p1_ref[...] = e1 / den
    mk_ref[...] = _uniform_at(2, j) < jnp.float32(0.5)


@functools.cache
def _build_tc_kernel():
    plane = jax.ShapeDtypeStruct((_B, _S), jnp.float32)
    return pl.pallas_call(
        _tc_body,
        out_shape=[plane, plane, plane, plane,
                   jax.ShapeDtypeStruct((_B, _S), jnp.bool_)],
    )


def kernel(x):
    B, S = x.shape
    l0, l1, p0, p1, mask = _build_tc_kernel()()
    ch, mi = _build_sc_kernel()(x, p0, p1)
    logits = jnp.stack([l0, l1], axis=-1)
    policy = jnp.stack([p0, p1], axis=-1)
    chosen = ch.reshape(B, S, 1)
    masked = mi
    return logits, policy, chosen, mask, masked
